# baseline (speedup 1.0000x reference)
import functools

import jax
import jax.numpy as jnp
from jax import lax
from jax.experimental import pallas as pl
from jax.experimental.pallas import tpu as pltpu
from jax.experimental.pallas import tpu_sc as plsc

_N = 10000
_E = 320000
_L = 16

_NT = 32
_NPT = 320
_NPAD = _NT * _NPT

_CH = 3200
_NCHUNK = _E // _CH
_SEG = _CH + 64
_ECAP = _NCHUNK * _SEG
_CSEG = _NCHUNK * _L


def _iota():
    return lax.broadcasted_iota(jnp.int32, (_L,), 0)


_NBUF = 8


def _phase_b(D, G, feat, ldl, lsrc, lw, out, lbd, lbs, lbw, rows2, acc1, cntv, gsem, wid, lo):
    del G
    neg = jnp.float32(-jnp.inf)
    nacc = (_NPT + 1) * D

    def initk(i, carry):
        acc1[pl.ds(i * _L, _L)] = jnp.full((_L,), neg, jnp.float32)
        return carry

    lax.fori_loop(0, nacc // _L, initk, 0)

    def seg_body(c, carry):
        cnt = cntv[pl.ds(c * _L, _L)][0]

        @pl.when(cnt > 0)
        def _():
            sbase = wid * _ECAP + c * _SEG
            pltpu.sync_copy(ldl.at[pl.ds(sbase, _SEG)], lbd)
            pltpu.sync_copy(lsrc.at[pl.ds(sbase, _SEG)], lbs)
            pltpu.sync_copy(lw.at[pl.ds(sbase, _SEG)], lbw)
            ng = (cnt + _L - 1) // _L

            def fire(gb, slot):
                pltpu.async_copy(
                    feat.at[lbs.at[pl.ds(gb * _L, _L)]],
                    rows2.at[pl.ds(slot * _L, _L)],
                    gsem.at[slot])

            def wait(gb, slot):
                pltpu.make_async_copy(
                    feat.at[lbs.at[pl.ds(gb * _L, _L)]],
                    rows2.at[pl.ds(slot * _L, _L)],
                    gsem.at[slot]).wait()

            for b in range(_NBUF - 1):
                @pl.when(b < ng)
                def _(b=b):
                    fire(b, b)

            def gb_body(gb, c3):
                s = gb % _NBUF
                wait(gb, s)

                @pl.when(gb + _NBUF - 1 < ng)
                def _():
                    fire(gb + _NBUF - 1, (gb + _NBUF - 1) % _NBUF)

                goff = gb * _L
                dlv = lbd[pl.ds(goff, _L)]
                wlv = lbw[pl.ds(goff, _L)]
                dav = dlv * D
                for l in range(_L):
                    wl = wlv[l]
                    av0 = dav[l] + _iota()
                    rr = s * _L + l
                    for half in range(D // 128):
                        avs, accs, rws = [], [], []
                        for kk in range(8):
                            k = half * 8 + kk
                            av = av0 + k * _L
                            avs.append(av)
                            accs.append(plsc.load_gather(acc1, [av]))
                            rws.append(rows2[rr, pl.ds(k * _L, _L)])
                        for kk in range(8):
                            plsc.store_scatter(
                                acc1, [avs[kk]],
                                jnp.maximum(accs[kk], rws[kk] * wl))
                return c3

            lax.fori_loop(0, ng, gb_body, 0)

        return carry

    lax.fori_loop(0, _NCHUNK, seg_body, 0)

    def fin(i, carry):
        sl = pl.ds(i * _L, _L)
        v = acc1[sl]
        acc1[sl] = jnp.where(v == neg, jnp.float32(0.0), v)
        return carry

    lax.fori_loop(0, (_NPT * D) // _L, fin, 0)
    pltpu.sync_copy(acc1.at[pl.ds(0, _NPT * D)], out.at[pl.ds(lo * D, _NPT * D)])


@functools.lru_cache(maxsize=None)
def _make_build_segmax(D, G):
    mesh = plsc.VectorSubcoreMesh(core_axis_name="c", subcore_axis_name="s")

    @functools.partial(
        pl.kernel,
        mesh=mesh,
        compiler_params=pltpu.CompilerParams(needs_layout_passes=False),
        out_type=(
            jax.ShapeDtypeStruct((_NPAD * D,), jnp.float32),
            jax.ShapeDtypeStruct((_NT * _ECAP,), jnp.int32),
            jax.ShapeDtypeStruct((_NT * _ECAP,), jnp.int32),
            jax.ShapeDtypeStruct((_NT * _ECAP,), jnp.float32),
            jax.ShapeDtypeStruct((_NT * _CSEG,), jnp.int32),
        ),
        scratch_types=[
            pltpu.VMEM((2 * _CH,), jnp.int32),
            pltpu.VMEM((2 * _CH,), jnp.int32),
            pltpu.VMEM((2 * _CH,), jnp.float32),
            pltpu.VMEM((2 * _SEG,), jnp.int32),
            pltpu.VMEM((2 * _SEG,), jnp.int32),
            pltpu.VMEM((2 * _SEG,), jnp.float32),
            pltpu.VMEM((_CSEG,), jnp.int32),
            pltpu.VMEM((_SEG,), jnp.int32),
            pltpu.VMEM((_SEG,), jnp.int32),
            pltpu.VMEM((_SEG,), jnp.float32),
            pltpu.VMEM((_NBUF * _L, D), jnp.float32),
            pltpu.VMEM(((_NPT + 1) * D,), jnp.float32),
            pltpu.SemaphoreType.DMA((2,)),
            pltpu.SemaphoreType.DMA((2,)),
            pltpu.SemaphoreType.DMA((_NBUF,)),
        ],
    )
    def build_segmax(feat, srcg, dstg, wg,
                     out, ldl, lsrc, lw, cnts,
                     ind, ins, inw, outd, outs, outw, cntv,
                     lbd, lbs, lbw, rows2, acc1,
                     isem, osem, gsem):
        wid = lax.axis_index("s") * 2 + lax.axis_index("c")
        lo = wid * _NPT

        def clr(i, carry):
            sl = pl.ds(i * _L, _L)
            outd[sl] = jnp.full((_L,), _NPT, jnp.int32)
            outs[sl] = jnp.zeros((_L,), jnp.int32)
            outw[sl] = jnp.zeros((_L,), jnp.float32)
            return carry

        lax.fori_loop(0, (2 * _SEG) // _L, clr, 0)

        def fire_in(c, slot):
            base = c * _CH
            off = slot * _CH
            pltpu.async_copy(dstg.at[pl.ds(base, _CH)], ind.at[pl.ds(off, _CH)], isem.at[slot])
            pltpu.async_copy(srcg.at[pl.ds(base, _CH)], ins.at[pl.ds(off, _CH)], isem.at[slot])
            pltpu.async_copy(wg.at[pl.ds(base, _CH)], inw.at[pl.ds(off, _CH)], isem.at[slot])

        def wait_in(c, slot):
            base = c * _CH
            off = slot * _CH
            pltpu.make_async_copy(dstg.at[pl.ds(base, _CH)], ind.at[pl.ds(off, _CH)], isem.at[slot]).wait()
            pltpu.make_async_copy(srcg.at[pl.ds(base, _CH)], ins.at[pl.ds(off, _CH)], isem.at[slot]).wait()
            pltpu.make_async_copy(wg.at[pl.ds(base, _CH)], inw.at[pl.ds(off, _CH)], isem.at[slot]).wait()

        def fire_out(c, slot):
            hb = wid * _ECAP + c * _SEG
            off = slot * _SEG
            pltpu.async_copy(outd.at[pl.ds(off, _SEG)], ldl.at[pl.ds(hb, _SEG)], osem.at[slot])
            pltpu.async_copy(outs.at[pl.ds(off, _SEG)], lsrc.at[pl.ds(hb, _SEG)], osem.at[slot])
            pltpu.async_copy(outw.at[pl.ds(off, _SEG)], lw.at[pl.ds(hb, _SEG)], osem.at[slot])

        def wait_out(c, slot):
            hb = wid * _ECAP + c * _SEG
            off = slot * _SEG
            pltpu.make_async_copy(outd.at[pl.ds(off, _SEG)], ldl.at[pl.ds(hb, _SEG)], osem.at[slot]).wait()
            pltpu.make_async_copy(outs.at[pl.ds(off, _SEG)], lsrc.at[pl.ds(hb, _SEG)], osem.at[slot]).wait()
            pltpu.make_async_copy(outw.at[pl.ds(off, _SEG)], lw.at[pl.ds(hb, _SEG)], osem.at[slot]).wait()

        fire_in(0, 0)

        def chunk(c, carry):
            cur = c % 2
            wait_in(c, cur)

            @pl.when(c + 1 < _NCHUNK)
            def _():
                fire_in(c + 1, 1 - cur)

            @pl.when(c >= 2)
            def _():
                wait_out(c - 2, cur)

            ibase = cur * _CH
            obase = cur * _SEG

            def filt(i, lptrv):
                lp = lptrv
                for u in range(4):
                    sl = pl.ds(ibase + i * (4 * _L) + u * _L, _L)
                    dv = ind[sl]
                    sv = ins[sl]
                    wv = inw[sl]
                    m = (dv >= lo) & (dv < lo + _NPT)
                    mi = m.astype(jnp.int32)
                    pos = obase + lp + jnp.cumsum(mi) - 1
                    plsc.store_scatter(outd, [pos], dv - lo, mask=m)
                    plsc.store_scatter(outs, [pos], sv, mask=m)
                    plsc.store_scatter(outw, [pos], wv, mask=m)
                    lp = lp + plsc.all_reduce_population_count(m)
                return lp

            lptrv = lax.fori_loop(0, _CH // (4 * _L), filt, jnp.zeros((_L,), jnp.int32))

            for pv in range(2):
                ppos = obase + lptrv + pv * _L + _iota()
                plsc.store_scatter(outd, [ppos], jnp.full((_L,), _NPT, jnp.int32))
                plsc.store_scatter(outs, [ppos], jnp.zeros((_L,), jnp.int32))
                plsc.store_scatter(outw, [ppos], jnp.zeros((_L,), jnp.float32))

            cntv[pl.ds(c * _L, _L)] = lptrv
            fire_out(c, cur)
            return carry

        lax.fori_loop(0, _NCHUNK, chunk, 0)
        wait_out(_NCHUNK - 2, (_NCHUNK - 2) % 2)
        wait_out(_NCHUNK - 1, (_NCHUNK - 1) % 2)
        pltpu.sync_copy(cntv, cnts.at[pl.ds(wid * _CSEG, _CSEG)])

        _phase_b(D, G, feat, ldl, lsrc, lw, out, lbd, lbs, lbw, rows2, acc1, cntv, gsem, wid, lo)

    return build_segmax


@functools.lru_cache(maxsize=None)
def _make_reuse_segmax(D, G):
    mesh = plsc.VectorSubcoreMesh(core_axis_name="c", subcore_axis_name="s")

    @functools.partial(
        pl.kernel,
        mesh=mesh,
        compiler_params=pltpu.CompilerParams(needs_layout_passes=False),
        out_type=jax.ShapeDtypeStruct((_NPAD * D,), jnp.float32),
        scratch_types=[
            pltpu.VMEM((_CSEG,), jnp.int32),
            pltpu.VMEM((_SEG,), jnp.int32),
            pltpu.VMEM((_SEG,), jnp.int32),
            pltpu.VMEM((_SEG,), jnp.float32),
            pltpu.VMEM((_NBUF * _L, D), jnp.float32),
            pltpu.VMEM(((_NPT + 1) * D,), jnp.float32),
            pltpu.SemaphoreType.DMA((_NBUF,)),
        ],
    )
    def reuse_segmax(feat, ldl, lsrc, lw, cnts, out,
                     cntv, lbd, lbs, lbw, rows2, acc1, gsem):
        wid = lax.axis_index("s") * 2 + lax.axis_index("c")
        lo = wid * _NPT
        pltpu.sync_copy(cnts.at[pl.ds(wid * _CSEG, _CSEG)], cntv)
        _phase_b(D, G, feat, ldl, lsrc, lw, out, lbd, lbs, lbw, rows2, acc1, cntv, gsem, wid, lo)

    return reuse_segmax


_BR = 1000


def _tc1_body(agg_ref, x_ref, wr_ref, b_ref, wt_ref, o_ref):
    h = (jnp.dot(agg_ref[...], wr_ref[...], preferred_element_type=jnp.float32)
         + jnp.dot(x_ref[...], wt_ref[...], preferred_element_type=jnp.float32)
         + b_ref[...])
    o_ref[...] = jnp.maximum(h, 0.0)


def _tc1(agg, x, wrT, b, wtT):
    DIN, DH = wrT.shape
    return pl.pallas_call(
        _tc1_body,
        grid=(_N // _BR,),
        in_specs=[
            pl.BlockSpec((_BR, DIN), lambda i: (i, 0)),
            pl.BlockSpec((_BR, DIN), lambda i: (i, 0)),
            pl.BlockSpec((DIN, DH), lambda i: (0, 0)),
            pl.BlockSpec((1, DH), lambda i: (0, 0)),
            pl.BlockSpec((DIN, DH), lambda i: (0, 0)),
        ],
        out_specs=pl.BlockSpec((_BR, DH), lambda i: (i, 0)),
        out_shape=jax.ShapeDtypeStruct((_N, DH), jnp.float32),
    )(agg, x, wrT, b.reshape(1, DH), wtT)


def _tc2_body(agg_ref, h_ref, wr_ref, b_ref, wt_ref, wl_ref, bl_ref, o_ref):
    h = (jnp.dot(agg_ref[...], wr_ref[...], preferred_element_type=jnp.float32)
         + jnp.dot(h_ref[...], wt_ref[...], preferred_element_type=jnp.float32)
         + b_ref[...])
    h = jnp.maximum(h, 0.0)
    o_ref[...] = (jnp.dot(h, wl_ref[...], preferred_element_type=jnp.float32)
                  + bl_ref[...])


def _tc2(agg, h1, wrT, b, wtT, wlT, bl):
    DH, DOUT = wlT.shape
    return pl.pallas_call(
        _tc2_body,
        grid=(_N // _BR,),
        in_specs=[
            pl.BlockSpec((_BR, DH), lambda i: (i, 0)),
            pl.BlockSpec((_BR, DH), lambda i: (i, 0)),
            pl.BlockSpec((DH, DH), lambda i: (0, 0)),
            pl.BlockSpec((1, DH), lambda i: (0, 0)),
            pl.BlockSpec((DH, DH), lambda i: (0, 0)),
            pl.BlockSpec((DH, DOUT), lambda i: (0, 0)),
            pl.BlockSpec((1, DOUT), lambda i: (0, 0)),
        ],
        out_specs=pl.BlockSpec((_BR, DOUT), lambda i: (i, 0)),
        out_shape=jax.ShapeDtypeStruct((_N, DOUT), jnp.float32),
    )(agg, h1, wrT, b.reshape(1, DH), wtT, wlT, bl.reshape(1, DOUT))


def kernel(x, edge_index, edge_attr, W1_rel, b1_rel, W1_root, W2_rel, b2_rel, W2_root, W_lin, b_lin):
    src = edge_index[0]
    dst = edge_index[1]
    agg1f, ldl, lsrc, lw, cnts = _make_build_segmax(128, 128)(x, src, dst, edge_attr)
    agg1 = agg1f.reshape(_NPAD, 128)[:_N]
    h1 = _tc1(agg1, x, W1_rel.T, b1_rel, W1_root.T)
    agg2 = _make_reuse_segmax(256, 64)(h1, ldl, lsrc, lw, cnts).reshape(_NPAD, 256)[:_N]
    out = _tc2(agg2, h1, W2_rel.T, b2_rel, W2_root.T, W_lin.T, b_lin)
    return out


# speedup vs baseline: 1.0003x; 1.0003x over previous
import functools

import jax
import jax.numpy as jnp
from jax import lax
from jax.experimental import pallas as pl
from jax.experimental.pallas import tpu as pltpu
from jax.experimental.pallas import tpu_sc as plsc

_N = 10000
_E = 320000
_L = 16

_NT = 32
_NPT = 320
_NPAD = _NT * _NPT

_CH = 3200
_NCHUNK = _E // _CH
_SEG = _CH + 64
_ECAP = _NCHUNK * _SEG
_CSEG = _NCHUNK * _L


def _iota():
    return lax.broadcasted_iota(jnp.int32, (_L,), 0)


_NBUF = 6


def _phase_b(D, G, feat, ldl, lsrc, lw, out, lbd, lbs, lbw, rows2, acc1, cntv, gsem, wid, lo):
    del G
    neg = jnp.float32(-jnp.inf)
    nacc = (_NPT + 1) * D

    def initk(i, carry):
        acc1[pl.ds(i * _L, _L)] = jnp.full((_L,), neg, jnp.float32)
        return carry

    lax.fori_loop(0, nacc // _L, initk, 0)

    def seg_body(c, carry):
        cnt = cntv[pl.ds(c * _L, _L)][0]

        @pl.when(cnt > 0)
        def _():
            sbase = wid * _ECAP + c * _SEG
            pltpu.sync_copy(ldl.at[pl.ds(sbase, _SEG)], lbd)
            pltpu.sync_copy(lsrc.at[pl.ds(sbase, _SEG)], lbs)
            pltpu.sync_copy(lw.at[pl.ds(sbase, _SEG)], lbw)
            ng = (cnt + _L - 1) // _L

            def fire(gb, slot):
                pltpu.async_copy(
                    feat.at[lbs.at[pl.ds(gb * _L, _L)]],
                    rows2.at[pl.ds(slot * _L, _L)],
                    gsem.at[slot])

            def wait(gb, slot):
                pltpu.make_async_copy(
                    feat.at[lbs.at[pl.ds(gb * _L, _L)]],
                    rows2.at[pl.ds(slot * _L, _L)],
                    gsem.at[slot]).wait()

            for b in range(_NBUF - 1):
                @pl.when(b < ng)
                def _(b=b):
                    fire(b, b)

            def gb_body(gb, c3):
                s = gb % _NBUF
                wait(gb, s)

                @pl.when(gb + _NBUF - 1 < ng)
                def _():
                    fire(gb + _NBUF - 1, (gb + _NBUF - 1) % _NBUF)

                goff = gb * _L
                dlv = lbd[pl.ds(goff, _L)]
                wlv = lbw[pl.ds(goff, _L)]
                dav = dlv * D
                for l in range(_L):
                    wl = wlv[l]
                    av0 = dav[l] + _iota()
                    rr = s * _L + l
                    for half in range(D // 128):
                        avs, accs, rws = [], [], []
                        for kk in range(8):
                            k = half * 8 + kk
                            av = av0 + k * _L
                            avs.append(av)
                            accs.append(plsc.load_gather(acc1, [av]))
                            rws.append(rows2[rr, pl.ds(k * _L, _L)])
                        for kk in range(8):
                            plsc.store_scatter(
                                acc1, [avs[kk]],
                                jnp.maximum(accs[kk], rws[kk] * wl))
                return c3

            lax.fori_loop(0, ng, gb_body, 0)

        return carry

    lax.fori_loop(0, _NCHUNK, seg_body, 0)

    def fin(i, carry):
        sl = pl.ds(i * _L, _L)
        v = acc1[sl]
        acc1[sl] = jnp.where(v == neg, jnp.float32(0.0), v)
        return carry

    lax.fori_loop(0, (_NPT * D) // _L, fin, 0)
    pltpu.sync_copy(acc1.at[pl.ds(0, _NPT * D)], out.at[pl.ds(lo * D, _NPT * D)])


@functools.lru_cache(maxsize=None)
def _make_build_segmax(D, G):
    mesh = plsc.VectorSubcoreMesh(core_axis_name="c", subcore_axis_name="s")

    @functools.partial(
        pl.kernel,
        mesh=mesh,
        compiler_params=pltpu.CompilerParams(needs_layout_passes=False),
        out_type=(
            jax.ShapeDtypeStruct((_NPAD * D,), jnp.float32),
            jax.ShapeDtypeStruct((_NT * _ECAP,), jnp.int32),
            jax.ShapeDtypeStruct((_NT * _ECAP,), jnp.int32),
            jax.ShapeDtypeStruct((_NT * _ECAP,), jnp.float32),
            jax.ShapeDtypeStruct((_NT * _CSEG,), jnp.int32),
        ),
        scratch_types=[
            pltpu.VMEM((2 * _CH,), jnp.int32),
            pltpu.VMEM((2 * _CH,), jnp.int32),
            pltpu.VMEM((2 * _CH,), jnp.float32),
            pltpu.VMEM((2 * _SEG,), jnp.int32),
            pltpu.VMEM((2 * _SEG,), jnp.int32),
            pltpu.VMEM((2 * _SEG,), jnp.float32),
            pltpu.VMEM((_CSEG,), jnp.int32),
            pltpu.VMEM((_SEG,), jnp.int32),
            pltpu.VMEM((_SEG,), jnp.int32),
            pltpu.VMEM((_SEG,), jnp.float32),
            pltpu.VMEM((_NBUF * _L, D), jnp.float32),
            pltpu.VMEM(((_NPT + 1) * D,), jnp.float32),
            pltpu.SemaphoreType.DMA((2,)),
            pltpu.SemaphoreType.DMA((2,)),
            pltpu.SemaphoreType.DMA((_NBUF,)),
        ],
    )
    def build_segmax(feat, srcg, dstg, wg,
                     out, ldl, lsrc, lw, cnts,
                     ind, ins, inw, outd, outs, outw, cntv,
                     lbd, lbs, lbw, rows2, acc1,
                     isem, osem, gsem):
        wid = lax.axis_index("s") * 2 + lax.axis_index("c")
        lo = wid * _NPT

        def clr(i, carry):
            sl = pl.ds(i * _L, _L)
            outd[sl] = jnp.full((_L,), _NPT, jnp.int32)
            outs[sl] = jnp.zeros((_L,), jnp.int32)
            outw[sl] = jnp.zeros((_L,), jnp.float32)
            return carry

        lax.fori_loop(0, (2 * _SEG) // _L, clr, 0)

        def fire_in(c, slot):
            base = c * _CH
            off = slot * _CH
            pltpu.async_copy(dstg.at[pl.ds(base, _CH)], ind.at[pl.ds(off, _CH)], isem.at[slot])
            pltpu.async_copy(srcg.at[pl.ds(base, _CH)], ins.at[pl.ds(off, _CH)], isem.at[slot])
            pltpu.async_copy(wg.at[pl.ds(base, _CH)], inw.at[pl.ds(off, _CH)], isem.at[slot])

        def wait_in(c, slot):
            base = c * _CH
            off = slot * _CH
            pltpu.make_async_copy(dstg.at[pl.ds(base, _CH)], ind.at[pl.ds(off, _CH)], isem.at[slot]).wait()
            pltpu.make_async_copy(srcg.at[pl.ds(base, _CH)], ins.at[pl.ds(off, _CH)], isem.at[slot]).wait()
            pltpu.make_async_copy(wg.at[pl.ds(base, _CH)], inw.at[pl.ds(off, _CH)], isem.at[slot]).wait()

        def fire_out(c, slot):
            hb = wid * _ECAP + c * _SEG
            off = slot * _SEG
            pltpu.async_copy(outd.at[pl.ds(off, _SEG)], ldl.at[pl.ds(hb, _SEG)], osem.at[slot])
            pltpu.async_copy(outs.at[pl.ds(off, _SEG)], lsrc.at[pl.ds(hb, _SEG)], osem.at[slot])
            pltpu.async_copy(outw.at[pl.ds(off, _SEG)], lw.at[pl.ds(hb, _SEG)], osem.at[slot])

        def wait_out(c, slot):
            hb = wid * _ECAP + c * _SEG
            off = slot * _SEG
            pltpu.make_async_copy(outd.at[pl.ds(off, _SEG)], ldl.at[pl.ds(hb, _SEG)], osem.at[slot]).wait()
            pltpu.make_async_copy(outs.at[pl.ds(off, _SEG)], lsrc.at[pl.ds(hb, _SEG)], osem.at[slot]).wait()
            pltpu.make_async_copy(outw.at[pl.ds(off, _SEG)], lw.at[pl.ds(hb, _SEG)], osem.at[slot]).wait()

        fire_in(0, 0)

        def chunk(c, carry):
            cur = c % 2
            wait_in(c, cur)

            @pl.when(c + 1 < _NCHUNK)
            def _():
                fire_in(c + 1, 1 - cur)

            @pl.when(c >= 2)
            def _():
                wait_out(c - 2, cur)

            ibase = cur * _CH
            obase = cur * _SEG

            def filt(i, lptrv):
                lp = lptrv
                for u in range(4):
                    sl = pl.ds(ibase + i * (4 * _L) + u * _L, _L)
                    dv = ind[sl]
                    sv = ins[sl]
                    wv = inw[sl]
                    m = (dv >= lo) & (dv < lo + _NPT)
                    mi = m.astype(jnp.int32)
                    pos = obase + lp + jnp.cumsum(mi) - 1
                    plsc.store_scatter(outd, [pos], dv - lo, mask=m)
                    plsc.store_scatter(outs, [pos], sv, mask=m)
                    plsc.store_scatter(outw, [pos], wv, mask=m)
                    lp = lp + plsc.all_reduce_population_count(m)
                return lp

            lptrv = lax.fori_loop(0, _CH // (4 * _L), filt, jnp.zeros((_L,), jnp.int32))

            for pv in range(2):
                ppos = obase + lptrv + pv * _L + _iota()
                plsc.store_scatter(outd, [ppos], jnp.full((_L,), _NPT, jnp.int32))
                plsc.store_scatter(outs, [ppos], jnp.zeros((_L,), jnp.int32))
                plsc.store_scatter(outw, [ppos], jnp.zeros((_L,), jnp.float32))

            cntv[pl.ds(c * _L, _L)] = lptrv
            fire_out(c, cur)
            return carry

        lax.fori_loop(0, _NCHUNK, chunk, 0)
        wait_out(_NCHUNK - 2, (_NCHUNK - 2) % 2)
        wait_out(_NCHUNK - 1, (_NCHUNK - 1) % 2)
        pltpu.sync_copy(cntv, cnts.at[pl.ds(wid * _CSEG, _CSEG)])

        _phase_b(D, G, feat, ldl, lsrc, lw, out, lbd, lbs, lbw, rows2, acc1, cntv, gsem, wid, lo)

    return build_segmax


@functools.lru_cache(maxsize=None)
def _make_reuse_segmax(D, G):
    mesh = plsc.VectorSubcoreMesh(core_axis_name="c", subcore_axis_name="s")

    @functools.partial(
        pl.kernel,
        mesh=mesh,
        compiler_params=pltpu.CompilerParams(needs_layout_passes=False),
        out_type=jax.ShapeDtypeStruct((_NPAD * D,), jnp.float32),
        scratch_types=[
            pltpu.VMEM((_CSEG,), jnp.int32),
            pltpu.VMEM((_SEG,), jnp.int32),
            pltpu.VMEM((_SEG,), jnp.int32),
            pltpu.VMEM((_SEG,), jnp.float32),
            pltpu.VMEM((_NBUF * _L, D), jnp.float32),
            pltpu.VMEM(((_NPT + 1) * D,), jnp.float32),
            pltpu.SemaphoreType.DMA((_NBUF,)),
        ],
    )
    def reuse_segmax(feat, ldl, lsrc, lw, cnts, out,
                     cntv, lbd, lbs, lbw, rows2, acc1, gsem):
        wid = lax.axis_index("s") * 2 + lax.axis_index("c")
        lo = wid * _NPT
        pltpu.sync_copy(cnts.at[pl.ds(wid * _CSEG, _CSEG)], cntv)
        _phase_b(D, G, feat, ldl, lsrc, lw, out, lbd, lbs, lbw, rows2, acc1, cntv, gsem, wid, lo)

    return reuse_segmax


_BR = 1000


def _tc1_body(agg_ref, x_ref, wr_ref, b_ref, wt_ref, o_ref):
    h = (jnp.dot(agg_ref[...], wr_ref[...], preferred_element_type=jnp.float32)
         + jnp.dot(x_ref[...], wt_ref[...], preferred_element_type=jnp.float32)
         + b_ref[...])
    o_ref[...] = jnp.maximum(h, 0.0)


def _tc1(agg, x, wrT, b, wtT):
    DIN, DH = wrT.shape
    return pl.pallas_call(
        _tc1_body,
        grid=(_N // _BR,),
        in_specs=[
            pl.BlockSpec((_BR, DIN), lambda i: (i, 0)),
            pl.BlockSpec((_BR, DIN), lambda i: (i, 0)),
            pl.BlockSpec((DIN, DH), lambda i: (0, 0)),
            pl.BlockSpec((1, DH), lambda i: (0, 0)),
            pl.BlockSpec((DIN, DH), lambda i: (0, 0)),
        ],
        out_specs=pl.BlockSpec((_BR, DH), lambda i: (i, 0)),
        out_shape=jax.ShapeDtypeStruct((_N, DH), jnp.float32),
    )(agg, x, wrT, b.reshape(1, DH), wtT)


def _tc2_body(agg_ref, h_ref, wr_ref, b_ref, wt_ref, wl_ref, bl_ref, o_ref):
    h = (jnp.dot(agg_ref[...], wr_ref[...], preferred_element_type=jnp.float32)
         + jnp.dot(h_ref[...], wt_ref[...], preferred_element_type=jnp.float32)
         + b_ref[...])
    h = jnp.maximum(h, 0.0)
    o_ref[...] = (jnp.dot(h, wl_ref[...], preferred_element_type=jnp.float32)
                  + bl_ref[...])


def _tc2(agg, h1, wrT, b, wtT, wlT, bl):
    DH, DOUT = wlT.shape
    return pl.pallas_call(
        _tc2_body,
        grid=(_N // _BR,),
        in_specs=[
            pl.BlockSpec((_BR, DH), lambda i: (i, 0)),
            pl.BlockSpec((_BR, DH), lambda i: (i, 0)),
            pl.BlockSpec((DH, DH), lambda i: (0, 0)),
            pl.BlockSpec((1, DH), lambda i: (0, 0)),
            pl.BlockSpec((DH, DH), lambda i: (0, 0)),
            pl.BlockSpec((DH, DOUT), lambda i: (0, 0)),
            pl.BlockSpec((1, DOUT), lambda i: (0, 0)),
        ],
        out_specs=pl.BlockSpec((_BR, DOUT), lambda i: (i, 0)),
        out_shape=jax.ShapeDtypeStruct((_N, DOUT), jnp.float32),
    )(agg, h1, wrT, b.reshape(1, DH), wtT, wlT, bl.reshape(1, DOUT))


def kernel(x, edge_index, edge_attr, W1_rel, b1_rel, W1_root, W2_rel, b2_rel, W2_root, W_lin, b_lin):
    src = edge_index[0]
    dst = edge_index[1]
    agg1f, ldl, lsrc, lw, cnts = _make_build_segmax(128, 128)(x, src, dst, edge_attr)
    agg1 = agg1f.reshape(_NPAD, 128)[:_N]
    h1 = _tc1(agg1, x, W1_rel.T, b1_rel, W1_root.T)
    agg2 = _make_reuse_segmax(256, 64)(h1, ldl, lsrc, lw, cnts).reshape(_NPAD, 256)[:_N]
    out = _tc2(agg2, h1, W2_rel.T, b2_rel, W2_root.T, W_lin.T, b_lin)
    return out


# speedup vs baseline: 1.0025x; 1.0022x over previous
import functools

import jax
import jax.numpy as jnp
from jax import lax
from jax.experimental import pallas as pl
from jax.experimental.pallas import tpu as pltpu
from jax.experimental.pallas import tpu_sc as plsc

_N = 10000
_E = 320000
_L = 16

_NT = 32
_NPT = 320
_NPAD = _NT * _NPT

_CH = 3200
_NCHUNK = _E // _CH
_SEG = _CH + 64
_ECAP = _NCHUNK * _SEG
_CSEG = _NCHUNK * _L


def _iota():
    return lax.broadcasted_iota(jnp.int32, (_L,), 0)


def _phase_b(D, NB, feat, ldl, lsrc, lw, out, lbd, lbs, lbw, rows2, acc1, cntv, gsem, lsem, wid, lo):
    neg = jnp.float32(-jnp.inf)
    nacc = (_NPT + 1) * D

    def initk(i, carry):
        acc1[pl.ds(i * _L, _L)] = jnp.full((_L,), neg, jnp.float32)
        return carry

    lax.fori_loop(0, nacc // _L, initk, 0)

    def stage(c, slot):
        sbase = wid * _ECAP + c * _SEG
        off = slot * _SEG
        pltpu.async_copy(ldl.at[pl.ds(sbase, _SEG)], lbd.at[pl.ds(off, _SEG)], lsem.at[slot])
        pltpu.async_copy(lsrc.at[pl.ds(sbase, _SEG)], lbs.at[pl.ds(off, _SEG)], lsem.at[slot])
        pltpu.async_copy(lw.at[pl.ds(sbase, _SEG)], lbw.at[pl.ds(off, _SEG)], lsem.at[slot])

    def stage_wait(c, slot):
        sbase = wid * _ECAP + c * _SEG
        off = slot * _SEG
        pltpu.make_async_copy(ldl.at[pl.ds(sbase, _SEG)], lbd.at[pl.ds(off, _SEG)], lsem.at[slot]).wait()
        pltpu.make_async_copy(lsrc.at[pl.ds(sbase, _SEG)], lbs.at[pl.ds(off, _SEG)], lsem.at[slot]).wait()
        pltpu.make_async_copy(lw.at[pl.ds(sbase, _SEG)], lbw.at[pl.ds(off, _SEG)], lsem.at[slot]).wait()

    stage(0, 0)

    def seg_body(c, carry):
        sl0 = c % 2
        stage_wait(c, sl0)

        @pl.when(c + 1 < _NCHUNK)
        def _():
            stage(c + 1, 1 - sl0)

        cnt = cntv[pl.ds(c * _L, _L)][0]

        @pl.when(cnt > 0)
        def _():
            lb0 = sl0 * _SEG
            ng = (cnt + _L - 1) // _L

            def fire(gb, slot):
                pltpu.async_copy(
                    feat.at[lbs.at[pl.ds(lb0 + gb * _L, _L)]],
                    rows2.at[pl.ds(slot * _L, _L)],
                    gsem.at[slot])

            def wait(gb, slot):
                pltpu.make_async_copy(
                    feat.at[lbs.at[pl.ds(lb0 + gb * _L, _L)]],
                    rows2.at[pl.ds(slot * _L, _L)],
                    gsem.at[slot]).wait()

            for b in range(NB - 1):
                @pl.when(b < ng)
                def _(b=b):
                    fire(b, b)

            def gb_body(gb, c3):
                s = gb % NB
                wait(gb, s)

                @pl.when(gb + NB - 1 < ng)
                def _():
                    fire(gb + NB - 1, (gb + NB - 1) % NB)

                goff = lb0 + gb * _L
                dlv = lbd[pl.ds(goff, _L)]
                wlv = lbw[pl.ds(goff, _L)]
                dav = dlv * D
                for l in range(_L):
                    wl = wlv[l]
                    av0 = dav[l] + _iota()
                    rr = s * _L + l
                    for half in range(D // 128):
                        avs, accs, rws = [], [], []
                        for kk in range(8):
                            k = half * 8 + kk
                            av = av0 + k * _L
                            avs.append(av)
                            accs.append(plsc.load_gather(acc1, [av]))
                            rws.append(rows2[rr, pl.ds(k * _L, _L)])
                        for kk in range(8):
                            plsc.store_scatter(
                                acc1, [avs[kk]],
                                jnp.maximum(accs[kk], rws[kk] * wl))
                return c3

            lax.fori_loop(0, ng, gb_body, 0)

        return carry

    lax.fori_loop(0, _NCHUNK, seg_body, 0)

    def fin(i, carry):
        sl = pl.ds(i * _L, _L)
        v = acc1[sl]
        acc1[sl] = jnp.where(v == neg, jnp.float32(0.0), v)
        return carry

    lax.fori_loop(0, (_NPT * D) // _L, fin, 0)
    pltpu.sync_copy(acc1.at[pl.ds(0, _NPT * D)], out.at[pl.ds(lo * D, _NPT * D)])


@functools.lru_cache(maxsize=None)
def _make_build_segmax(D, G):
    mesh = plsc.VectorSubcoreMesh(core_axis_name="c", subcore_axis_name="s")

    @functools.partial(
        pl.kernel,
        mesh=mesh,
        compiler_params=pltpu.CompilerParams(needs_layout_passes=False),
        out_type=(
            jax.ShapeDtypeStruct((_NPAD * D,), jnp.float32),
            jax.ShapeDtypeStruct((_NT * _ECAP,), jnp.int32),
            jax.ShapeDtypeStruct((_NT * _ECAP,), jnp.int32),
            jax.ShapeDtypeStruct((_NT * _ECAP,), jnp.float32),
            jax.ShapeDtypeStruct((_NT * _CSEG,), jnp.int32),
        ),
        scratch_types=[
            pltpu.VMEM((2 * _CH,), jnp.int32),
            pltpu.VMEM((2 * _CH,), jnp.int32),
            pltpu.VMEM((2 * _CH,), jnp.float32),
            pltpu.VMEM((2 * _SEG,), jnp.int32),
            pltpu.VMEM((2 * _SEG,), jnp.int32),
            pltpu.VMEM((2 * _SEG,), jnp.float32),
            pltpu.VMEM((_CSEG,), jnp.int32),
            pltpu.VMEM((2 * _SEG,), jnp.int32),
            pltpu.VMEM((2 * _SEG,), jnp.int32),
            pltpu.VMEM((2 * _SEG,), jnp.float32),
            pltpu.VMEM((G * _L, D), jnp.float32),
            pltpu.VMEM(((_NPT + 1) * D,), jnp.float32),
            pltpu.SemaphoreType.DMA((2,)),
            pltpu.SemaphoreType.DMA((2,)),
            pltpu.SemaphoreType.DMA((G,)),
            pltpu.SemaphoreType.DMA((2,)),
        ],
    )
    def build_segmax(feat, srcg, dstg, wg,
                     out, ldl, lsrc, lw, cnts,
                     ind, ins, inw, outd, outs, outw, cntv,
                     lbd, lbs, lbw, rows2, acc1,
                     isem, osem, gsem, lsem):
        wid = lax.axis_index("s") * 2 + lax.axis_index("c")
        lo = wid * _NPT

        def clr(i, carry):
            sl = pl.ds(i * _L, _L)
            outd[sl] = jnp.full((_L,), _NPT, jnp.int32)
            outs[sl] = jnp.zeros((_L,), jnp.int32)
            outw[sl] = jnp.zeros((_L,), jnp.float32)
            return carry

        lax.fori_loop(0, (2 * _SEG) // _L, clr, 0)

        def fire_in(c, slot):
            base = c * _CH
            off = slot * _CH
            pltpu.async_copy(dstg.at[pl.ds(base, _CH)], ind.at[pl.ds(off, _CH)], isem.at[slot])
            pltpu.async_copy(srcg.at[pl.ds(base, _CH)], ins.at[pl.ds(off, _CH)], isem.at[slot])
            pltpu.async_copy(wg.at[pl.ds(base, _CH)], inw.at[pl.ds(off, _CH)], isem.at[slot])

        def wait_in(c, slot):
            base = c * _CH
            off = slot * _CH
            pltpu.make_async_copy(dstg.at[pl.ds(base, _CH)], ind.at[pl.ds(off, _CH)], isem.at[slot]).wait()
            pltpu.make_async_copy(srcg.at[pl.ds(base, _CH)], ins.at[pl.ds(off, _CH)], isem.at[slot]).wait()
            pltpu.make_async_copy(wg.at[pl.ds(base, _CH)], inw.at[pl.ds(off, _CH)], isem.at[slot]).wait()

        def fire_out(c, slot):
            hb = wid * _ECAP + c * _SEG
            off = slot * _SEG
            pltpu.async_copy(outd.at[pl.ds(off, _SEG)], ldl.at[pl.ds(hb, _SEG)], osem.at[slot])
            pltpu.async_copy(outs.at[pl.ds(off, _SEG)], lsrc.at[pl.ds(hb, _SEG)], osem.at[slot])
            pltpu.async_copy(outw.at[pl.ds(off, _SEG)], lw.at[pl.ds(hb, _SEG)], osem.at[slot])

        def wait_out(c, slot):
            hb = wid * _ECAP + c * _SEG
            off = slot * _SEG
            pltpu.make_async_copy(outd.at[pl.ds(off, _SEG)], ldl.at[pl.ds(hb, _SEG)], osem.at[slot]).wait()
            pltpu.make_async_copy(outs.at[pl.ds(off, _SEG)], lsrc.at[pl.ds(hb, _SEG)], osem.at[slot]).wait()
            pltpu.make_async_copy(outw.at[pl.ds(off, _SEG)], lw.at[pl.ds(hb, _SEG)], osem.at[slot]).wait()

        fire_in(0, 0)

        def chunk(c, carry):
            cur = c % 2
            wait_in(c, cur)

            @pl.when(c + 1 < _NCHUNK)
            def _():
                fire_in(c + 1, 1 - cur)

            @pl.when(c >= 2)
            def _():
                wait_out(c - 2, cur)

            ibase = cur * _CH
            obase = cur * _SEG

            def filt(i, lptrv):
                lp = lptrv
                for u in range(4):
                    sl = pl.ds(ibase + i * (4 * _L) + u * _L, _L)
                    dv = ind[sl]
                    sv = ins[sl]
                    wv = inw[sl]
                    m = (dv >= lo) & (dv < lo + _NPT)
                    mi = m.astype(jnp.int32)
                    pos = obase + lp + jnp.cumsum(mi) - 1
                    plsc.store_scatter(outd, [pos], dv - lo, mask=m)
                    plsc.store_scatter(outs, [pos], sv, mask=m)
                    plsc.store_scatter(outw, [pos], wv, mask=m)
                    lp = lp + plsc.all_reduce_population_count(m)
                return lp

            lptrv = lax.fori_loop(0, _CH // (4 * _L), filt, jnp.zeros((_L,), jnp.int32))

            for pv in range(2):
                ppos = obase + lptrv + pv * _L + _iota()
                plsc.store_scatter(outd, [ppos], jnp.full((_L,), _NPT, jnp.int32))
                plsc.store_scatter(outs, [ppos], jnp.zeros((_L,), jnp.int32))
                plsc.store_scatter(outw, [ppos], jnp.zeros((_L,), jnp.float32))

            cntv[pl.ds(c * _L, _L)] = lptrv
            fire_out(c, cur)
            return carry

        lax.fori_loop(0, _NCHUNK, chunk, 0)
        wait_out(_NCHUNK - 2, (_NCHUNK - 2) % 2)
        wait_out(_NCHUNK - 1, (_NCHUNK - 1) % 2)
        pltpu.sync_copy(cntv, cnts.at[pl.ds(wid * _CSEG, _CSEG)])

        _phase_b(D, G, feat, ldl, lsrc, lw, out, lbd, lbs, lbw, rows2, acc1, cntv, gsem, lsem, wid, lo)

    return build_segmax


@functools.lru_cache(maxsize=None)
def _make_reuse_segmax(D, G):
    mesh = plsc.VectorSubcoreMesh(core_axis_name="c", subcore_axis_name="s")

    @functools.partial(
        pl.kernel,
        mesh=mesh,
        compiler_params=pltpu.CompilerParams(needs_layout_passes=False),
        out_type=jax.ShapeDtypeStruct((_NPAD * D,), jnp.float32),
        scratch_types=[
            pltpu.VMEM((_CSEG,), jnp.int32),
            pltpu.VMEM((2 * _SEG,), jnp.int32),
            pltpu.VMEM((2 * _SEG,), jnp.int32),
            pltpu.VMEM((2 * _SEG,), jnp.float32),
            pltpu.VMEM((G * _L, D), jnp.float32),
            pltpu.VMEM(((_NPT + 1) * D,), jnp.float32),
            pltpu.SemaphoreType.DMA((G,)),
            pltpu.SemaphoreType.DMA((2,)),
        ],
    )
    def reuse_segmax(feat, ldl, lsrc, lw, cnts, out,
                     cntv, lbd, lbs, lbw, rows2, acc1, gsem, lsem):
        wid = lax.axis_index("s") * 2 + lax.axis_index("c")
        lo = wid * _NPT
        pltpu.sync_copy(cnts.at[pl.ds(wid * _CSEG, _CSEG)], cntv)
        _phase_b(D, G, feat, ldl, lsrc, lw, out, lbd, lbs, lbw, rows2, acc1, cntv, gsem, lsem, wid, lo)

    return reuse_segmax


_BR = 1000


def _tc1_body(agg_ref, x_ref, wr_ref, b_ref, wt_ref, o_ref):
    h = (jnp.dot(agg_ref[...], wr_ref[...], preferred_element_type=jnp.float32)
         + jnp.dot(x_ref[...], wt_ref[...], preferred_element_type=jnp.float32)
         + b_ref[...])
    o_ref[...] = jnp.maximum(h, 0.0)


def _tc1(agg, x, wrT, b, wtT):
    DIN, DH = wrT.shape
    return pl.pallas_call(
        _tc1_body,
        grid=(_N // _BR,),
        in_specs=[
            pl.BlockSpec((_BR, DIN), lambda i: (i, 0)),
            pl.BlockSpec((_BR, DIN), lambda i: (i, 0)),
            pl.BlockSpec((DIN, DH), lambda i: (0, 0)),
            pl.BlockSpec((1, DH), lambda i: (0, 0)),
            pl.BlockSpec((DIN, DH), lambda i: (0, 0)),
        ],
        out_specs=pl.BlockSpec((_BR, DH), lambda i: (i, 0)),
        out_shape=jax.ShapeDtypeStruct((_N, DH), jnp.float32),
    )(agg, x, wrT, b.reshape(1, DH), wtT)


def _tc2_body(agg_ref, h_ref, wr_ref, b_ref, wt_ref, wl_ref, bl_ref, o_ref):
    h = (jnp.dot(agg_ref[...], wr_ref[...], preferred_element_type=jnp.float32)
         + jnp.dot(h_ref[...], wt_ref[...], preferred_element_type=jnp.float32)
         + b_ref[...])
    h = jnp.maximum(h, 0.0)
    o_ref[...] = (jnp.dot(h, wl_ref[...], preferred_element_type=jnp.float32)
                  + bl_ref[...])


def _tc2(agg, h1, wrT, b, wtT, wlT, bl):
    DH, DOUT = wlT.shape
    return pl.pallas_call(
        _tc2_body,
        grid=(_N // _BR,),
        in_specs=[
            pl.BlockSpec((_BR, DH), lambda i: (i, 0)),
            pl.BlockSpec((_BR, DH), lambda i: (i, 0)),
            pl.BlockSpec((DH, DH), lambda i: (0, 0)),
            pl.BlockSpec((1, DH), lambda i: (0, 0)),
            pl.BlockSpec((DH, DH), lambda i: (0, 0)),
            pl.BlockSpec((DH, DOUT), lambda i: (0, 0)),
            pl.BlockSpec((1, DOUT), lambda i: (0, 0)),
        ],
        out_specs=pl.BlockSpec((_BR, DOUT), lambda i: (i, 0)),
        out_shape=jax.ShapeDtypeStruct((_N, DOUT), jnp.float32),
    )(agg, h1, wrT, b.reshape(1, DH), wtT, wlT, bl.reshape(1, DOUT))


def kernel(x, edge_index, edge_attr, W1_rel, b1_rel, W1_root, W2_rel, b2_rel, W2_root, W_lin, b_lin):
    src = edge_index[0]
    dst = edge_index[1]
    agg1f, ldl, lsrc, lw, cnts = _make_build_segmax(128, 6)(x, src, dst, edge_attr)
    agg1 = agg1f.reshape(_NPAD, 128)[:_N]
    h1 = _tc1(agg1, x, W1_rel.T, b1_rel, W1_root.T)
    agg2 = _make_reuse_segmax(256, 5)(h1, ldl, lsrc, lw, cnts).reshape(_NPAD, 256)[:_N]
    out = _tc2(agg2, h1, W2_rel.T, b2_rel, W2_root.T, W_lin.T, b_lin)
    return out


# speedup vs baseline: 1.1935x; 1.1906x over previous
import functools

import jax
import jax.numpy as jnp
from jax import lax
from jax.experimental import pallas as pl
from jax.experimental.pallas import tpu as pltpu
from jax.experimental.pallas import tpu_sc as plsc

_N = 10000
_E = 320000
_L = 16

_NT = 32
_NPT = 320
_NPAD = _NT * _NPT

_CH = 3200
_NCHUNK = _E // _CH
_SEG = _CH + 64
_ECAP = _NCHUNK * _SEG
_CSEG = _NCHUNK * _L


def _iota():
    return lax.broadcasted_iota(jnp.int32, (_L,), 0)


def _phase_b(D, NB, feat, ldl, lsrc, lw, out, lbd, lbs, lbw, rows2, acc1, cntv, gsem, lsem, wid, lo):
    neg = jnp.float32(-jnp.inf)
    nacc = (_NPT + 1) * D

    def initk(i, carry):
        acc1[pl.ds(i * _L, _L)] = jnp.full((_L,), neg, jnp.float32)
        return carry

    lax.fori_loop(0, nacc // _L, initk, 0)

    def stage(c, slot):
        sbase = wid * _ECAP + c * _SEG
        off = slot * _SEG
        pltpu.async_copy(ldl.at[pl.ds(sbase, _SEG)], lbd.at[pl.ds(off, _SEG)], lsem.at[slot])
        pltpu.async_copy(lsrc.at[pl.ds(sbase, _SEG)], lbs.at[pl.ds(off, _SEG)], lsem.at[slot])
        pltpu.async_copy(lw.at[pl.ds(sbase, _SEG)], lbw.at[pl.ds(off, _SEG)], lsem.at[slot])

    def stage_wait(c, slot):
        sbase = wid * _ECAP + c * _SEG
        off = slot * _SEG
        pltpu.make_async_copy(ldl.at[pl.ds(sbase, _SEG)], lbd.at[pl.ds(off, _SEG)], lsem.at[slot]).wait()
        pltpu.make_async_copy(lsrc.at[pl.ds(sbase, _SEG)], lbs.at[pl.ds(off, _SEG)], lsem.at[slot]).wait()
        pltpu.make_async_copy(lw.at[pl.ds(sbase, _SEG)], lbw.at[pl.ds(off, _SEG)], lsem.at[slot]).wait()

    stage(0, 0)

    def seg_body(c, carry):
        sl0 = c % 2
        stage_wait(c, sl0)

        @pl.when(c + 1 < _NCHUNK)
        def _():
            stage(c + 1, 1 - sl0)

        cnt = cntv[pl.ds(c * _L, _L)][0]

        @pl.when(cnt > 0)
        def _():
            lb0 = sl0 * _SEG
            ng = (cnt + _L - 1) // _L

            def fire(gb, slot):
                pltpu.async_copy(
                    feat.at[lbs.at[pl.ds(lb0 + gb * _L, _L)]],
                    rows2.at[pl.ds(slot * _L, _L)],
                    gsem.at[slot])

            def wait(gb, slot):
                pltpu.make_async_copy(
                    feat.at[lbs.at[pl.ds(lb0 + gb * _L, _L)]],
                    rows2.at[pl.ds(slot * _L, _L)],
                    gsem.at[slot]).wait()

            for b in range(NB - 1):
                @pl.when(b < ng)
                def _(b=b):
                    fire(b, b)

            def gb_body(gb, c3):
                s = gb % NB
                wait(gb, s)

                @pl.when(gb + NB - 1 < ng)
                def _():
                    fire(gb + NB - 1, (gb + NB - 1) % NB)

                goff = lb0 + gb * _L
                dlv = lbd[pl.ds(goff, _L)]
                wlv = lbw[pl.ds(goff, _L)]
                dav = dlv * D
                for l in range(_L):
                    wl = wlv[l]
                    av0 = dav[l] + _iota()
                    rr = s * _L + l
                    for half in range(D // 128):
                        avs, accs, rws = [], [], []
                        for kk in range(8):
                            k = half * 8 + kk
                            av = av0 + k * _L
                            avs.append(av)
                            accs.append(plsc.load_gather(acc1, [av]))
                        for j in range(4):
                            xw = rows2[rr, pl.ds(half * 64 + j * _L, _L)]
                            x32 = plsc.bitcast(xw, jnp.bfloat16)
                            ra, rb = plsc.unpack(x32, format=plsc.PackFormat.INTERLEAVED)
                            rws.append(ra)
                            rws.append(rb)
                        for kk in range(8):
                            plsc.store_scatter(
                                acc1, [avs[kk]],
                                jnp.maximum(accs[kk], rws[kk] * wl))
                return c3

            lax.fori_loop(0, ng, gb_body, 0)

        return carry

    lax.fori_loop(0, _NCHUNK, seg_body, 0)

    def fin(i, carry):
        sl = pl.ds(i * _L, _L)
        v = acc1[sl]
        acc1[sl] = jnp.where(v == neg, jnp.float32(0.0), v)
        return carry

    lax.fori_loop(0, (_NPT * D) // _L, fin, 0)
    pltpu.sync_copy(acc1.at[pl.ds(0, _NPT * D)], out.at[pl.ds(lo * D, _NPT * D)])


@functools.lru_cache(maxsize=None)
def _make_build_segmax(D, G):
    mesh = plsc.VectorSubcoreMesh(core_axis_name="c", subcore_axis_name="s")

    @functools.partial(
        pl.kernel,
        mesh=mesh,
        compiler_params=pltpu.CompilerParams(needs_layout_passes=False, use_tc_tiling_on_sc=False),
        out_type=(
            jax.ShapeDtypeStruct((_NPAD * D,), jnp.float32),
            jax.ShapeDtypeStruct((_NT * _ECAP,), jnp.int32),
            jax.ShapeDtypeStruct((_NT * _ECAP,), jnp.int32),
            jax.ShapeDtypeStruct((_NT * _ECAP,), jnp.float32),
            jax.ShapeDtypeStruct((_NT * _CSEG,), jnp.int32),
        ),
        scratch_types=[
            pltpu.VMEM((2 * _CH,), jnp.int32),
            pltpu.VMEM((2 * _CH,), jnp.int32),
            pltpu.VMEM((2 * _CH,), jnp.float32),
            pltpu.VMEM((2 * _SEG,), jnp.int32),
            pltpu.VMEM((2 * _SEG,), jnp.int32),
            pltpu.VMEM((2 * _SEG,), jnp.float32),
            pltpu.VMEM((_CSEG,), jnp.int32),
            pltpu.VMEM((2 * _SEG,), jnp.int32),
            pltpu.VMEM((2 * _SEG,), jnp.int32),
            pltpu.VMEM((2 * _SEG,), jnp.float32),
            pltpu.VMEM((G * _L, D // 2), jnp.int32),
            pltpu.VMEM(((_NPT + 1) * D,), jnp.float32),
            pltpu.SemaphoreType.DMA((2,)),
            pltpu.SemaphoreType.DMA((2,)),
            pltpu.SemaphoreType.DMA((G,)),
            pltpu.SemaphoreType.DMA((2,)),
        ],
    )
    def build_segmax(feat, srcg, dstg, wg,
                     out, ldl, lsrc, lw, cnts,
                     ind, ins, inw, outd, outs, outw, cntv,
                     lbd, lbs, lbw, rows2, acc1,
                     isem, osem, gsem, lsem):
        wid = lax.axis_index("s") * 2 + lax.axis_index("c")
        lo = wid * _NPT

        def clr(i, carry):
            sl = pl.ds(i * _L, _L)
            outd[sl] = jnp.full((_L,), _NPT, jnp.int32)
            outs[sl] = jnp.zeros((_L,), jnp.int32)
            outw[sl] = jnp.zeros((_L,), jnp.float32)
            return carry

        lax.fori_loop(0, (2 * _SEG) // _L, clr, 0)

        def fire_in(c, slot):
            base = c * _CH
            off = slot * _CH
            pltpu.async_copy(dstg.at[pl.ds(base, _CH)], ind.at[pl.ds(off, _CH)], isem.at[slot])
            pltpu.async_copy(srcg.at[pl.ds(base, _CH)], ins.at[pl.ds(off, _CH)], isem.at[slot])
            pltpu.async_copy(wg.at[pl.ds(base, _CH)], inw.at[pl.ds(off, _CH)], isem.at[slot])

        def wait_in(c, slot):
            base = c * _CH
            off = slot * _CH
            pltpu.make_async_copy(dstg.at[pl.ds(base, _CH)], ind.at[pl.ds(off, _CH)], isem.at[slot]).wait()
            pltpu.make_async_copy(srcg.at[pl.ds(base, _CH)], ins.at[pl.ds(off, _CH)], isem.at[slot]).wait()
            pltpu.make_async_copy(wg.at[pl.ds(base, _CH)], inw.at[pl.ds(off, _CH)], isem.at[slot]).wait()

        def fire_out(c, slot):
            hb = wid * _ECAP + c * _SEG
            off = slot * _SEG
            pltpu.async_copy(outd.at[pl.ds(off, _SEG)], ldl.at[pl.ds(hb, _SEG)], osem.at[slot])
            pltpu.async_copy(outs.at[pl.ds(off, _SEG)], lsrc.at[pl.ds(hb, _SEG)], osem.at[slot])
            pltpu.async_copy(outw.at[pl.ds(off, _SEG)], lw.at[pl.ds(hb, _SEG)], osem.at[slot])

        def wait_out(c, slot):
            hb = wid * _ECAP + c * _SEG
            off = slot * _SEG
            pltpu.make_async_copy(outd.at[pl.ds(off, _SEG)], ldl.at[pl.ds(hb, _SEG)], osem.at[slot]).wait()
            pltpu.make_async_copy(outs.at[pl.ds(off, _SEG)], lsrc.at[pl.ds(hb, _SEG)], osem.at[slot]).wait()
            pltpu.make_async_copy(outw.at[pl.ds(off, _SEG)], lw.at[pl.ds(hb, _SEG)], osem.at[slot]).wait()

        fire_in(0, 0)

        def chunk(c, carry):
            cur = c % 2
            wait_in(c, cur)

            @pl.when(c + 1 < _NCHUNK)
            def _():
                fire_in(c + 1, 1 - cur)

            @pl.when(c >= 2)
            def _():
                wait_out(c - 2, cur)

            ibase = cur * _CH
            obase = cur * _SEG

            def filt(i, lptrv):
                lp = lptrv
                for u in range(4):
                    sl = pl.ds(ibase + i * (4 * _L) + u * _L, _L)
                    dv = ind[sl]
                    sv = ins[sl]
                    wv = inw[sl]
                    m = (dv >= lo) & (dv < lo + _NPT)
                    mi = m.astype(jnp.int32)
                    pos = obase + lp + jnp.cumsum(mi) - 1
                    plsc.store_scatter(outd, [pos], dv - lo, mask=m)
                    plsc.store_scatter(outs, [pos], sv, mask=m)
                    plsc.store_scatter(outw, [pos], wv, mask=m)
                    lp = lp + plsc.all_reduce_population_count(m)
                return lp

            lptrv = lax.fori_loop(0, _CH // (4 * _L), filt, jnp.zeros((_L,), jnp.int32))

            for pv in range(2):
                ppos = obase + lptrv + pv * _L + _iota()
                plsc.store_scatter(outd, [ppos], jnp.full((_L,), _NPT, jnp.int32))
                plsc.store_scatter(outs, [ppos], jnp.zeros((_L,), jnp.int32))
                plsc.store_scatter(outw, [ppos], jnp.zeros((_L,), jnp.float32))

            cntv[pl.ds(c * _L, _L)] = lptrv
            fire_out(c, cur)
            return carry

        lax.fori_loop(0, _NCHUNK, chunk, 0)
        wait_out(_NCHUNK - 2, (_NCHUNK - 2) % 2)
        wait_out(_NCHUNK - 1, (_NCHUNK - 1) % 2)
        pltpu.sync_copy(cntv, cnts.at[pl.ds(wid * _CSEG, _CSEG)])

        _phase_b(D, G, feat, ldl, lsrc, lw, out, lbd, lbs, lbw, rows2, acc1, cntv, gsem, lsem, wid, lo)

    return build_segmax


@functools.lru_cache(maxsize=None)
def _make_reuse_segmax(D, G):
    mesh = plsc.VectorSubcoreMesh(core_axis_name="c", subcore_axis_name="s")

    @functools.partial(
        pl.kernel,
        mesh=mesh,
        compiler_params=pltpu.CompilerParams(needs_layout_passes=False, use_tc_tiling_on_sc=False),
        out_type=jax.ShapeDtypeStruct((_NPAD * D,), jnp.float32),
        scratch_types=[
            pltpu.VMEM((_CSEG,), jnp.int32),
            pltpu.VMEM((2 * _SEG,), jnp.int32),
            pltpu.VMEM((2 * _SEG,), jnp.int32),
            pltpu.VMEM((2 * _SEG,), jnp.float32),
            pltpu.VMEM((G * _L, D // 2), jnp.int32),
            pltpu.VMEM(((_NPT + 1) * D,), jnp.float32),
            pltpu.SemaphoreType.DMA((G,)),
            pltpu.SemaphoreType.DMA((2,)),
        ],
    )
    def reuse_segmax(feat, ldl, lsrc, lw, cnts, out,
                     cntv, lbd, lbs, lbw, rows2, acc1, gsem, lsem):
        wid = lax.axis_index("s") * 2 + lax.axis_index("c")
        lo = wid * _NPT
        pltpu.sync_copy(cnts.at[pl.ds(wid * _CSEG, _CSEG)], cntv)
        _phase_b(D, G, feat, ldl, lsrc, lw, out, lbd, lbs, lbw, rows2, acc1, cntv, gsem, lsem, wid, lo)

    return reuse_segmax


_BR = 1000


def _tc1_body(agg_ref, x_ref, wr_ref, b_ref, wt_ref, o_ref):
    h = (jnp.dot(agg_ref[...], wr_ref[...], preferred_element_type=jnp.float32)
         + jnp.dot(x_ref[...], wt_ref[...], preferred_element_type=jnp.float32)
         + b_ref[...])
    o_ref[...] = jnp.maximum(h, 0.0)


def _tc1(agg, x, wrT, b, wtT):
    DIN, DH = wrT.shape
    return pl.pallas_call(
        _tc1_body,
        grid=(_N // _BR,),
        in_specs=[
            pl.BlockSpec((_BR, DIN), lambda i: (i, 0)),
            pl.BlockSpec((_BR, DIN), lambda i: (i, 0)),
            pl.BlockSpec((DIN, DH), lambda i: (0, 0)),
            pl.BlockSpec((1, DH), lambda i: (0, 0)),
            pl.BlockSpec((DIN, DH), lambda i: (0, 0)),
        ],
        out_specs=pl.BlockSpec((_BR, DH), lambda i: (i, 0)),
        out_shape=jax.ShapeDtypeStruct((_N, DH), jnp.float32),
    )(agg, x, wrT, b.reshape(1, DH), wtT)


def _tc2_body(agg_ref, h_ref, wr_ref, b_ref, wt_ref, wl_ref, bl_ref, o_ref):
    h = (jnp.dot(agg_ref[...], wr_ref[...], preferred_element_type=jnp.float32)
         + jnp.dot(h_ref[...], wt_ref[...], preferred_element_type=jnp.float32)
         + b_ref[...])
    h = jnp.maximum(h, 0.0)
    o_ref[...] = (jnp.dot(h, wl_ref[...], preferred_element_type=jnp.float32)
                  + bl_ref[...])


def _tc2(agg, h1, wrT, b, wtT, wlT, bl):
    DH, DOUT = wlT.shape
    return pl.pallas_call(
        _tc2_body,
        grid=(_N // _BR,),
        in_specs=[
            pl.BlockSpec((_BR, DH), lambda i: (i, 0)),
            pl.BlockSpec((_BR, DH), lambda i: (i, 0)),
            pl.BlockSpec((DH, DH), lambda i: (0, 0)),
            pl.BlockSpec((1, DH), lambda i: (0, 0)),
            pl.BlockSpec((DH, DH), lambda i: (0, 0)),
            pl.BlockSpec((DH, DOUT), lambda i: (0, 0)),
            pl.BlockSpec((1, DOUT), lambda i: (0, 0)),
        ],
        out_specs=pl.BlockSpec((_BR, DOUT), lambda i: (i, 0)),
        out_shape=jax.ShapeDtypeStruct((_N, DOUT), jnp.float32),
    )(agg, h1, wrT, b.reshape(1, DH), wtT, wlT, bl.reshape(1, DOUT))


def _pack_perm(D):
    import numpy as np
    p = np.empty((D,), dtype=np.int32)
    for blk in range(D // 32):
        for i in range(16):
            p[blk * 32 + 2 * i] = blk * 32 + i
            p[blk * 32 + 2 * i + 1] = blk * 32 + 16 + i
    return p


def kernel(x, edge_index, edge_attr, W1_rel, b1_rel, W1_root, W2_rel, b2_rel, W2_root, W_lin, b_lin):
    src = edge_index[0]
    dst = edge_index[1]
    xb = jax.lax.bitcast_convert_type(
        x[:, _pack_perm(128)].astype(jnp.bfloat16).reshape(_N, 64, 2), jnp.int32)
    agg1f, ldl, lsrc, lw, cnts = _make_build_segmax(128, 6)(xb, src, dst, edge_attr)
    agg1 = agg1f.reshape(_NPAD, 128)[:_N]
    h1 = _tc1(agg1, x, W1_rel.T, b1_rel, W1_root.T)
    h1b = jax.lax.bitcast_convert_type(
        h1[:, _pack_perm(256)].astype(jnp.bfloat16).reshape(_N, 128, 2), jnp.int32)
    agg2 = _make_reuse_segmax(256, 5)(h1b, ldl, lsrc, lw, cnts).reshape(_NPAD, 256)[:_N]
    out = _tc2(agg2, h1, W2_rel.T, b2_rel, W2_root.T, W_lin.T, b_lin)
    return out
